# baseline (speedup 1.0000x reference)
import functools

import jax
import jax.numpy as jnp
from jax import lax
from jax.experimental import pallas as pl
from jax.experimental.pallas import tpu as pltpu
from jax.experimental.pallas import tpu_sc as plsc

N = 10000
E = 320000
D = 128
F1 = 16
NC = 2
NS = 16
NW = NC * NS
EPW = E // NW
CH = 80
NCHUNK = EPW // CH
NPAIR = (NCHUNK - 1) // 2

_mesh = plsc.VectorSubcoreMesh(
    core_axis_name="c", subcore_axis_name="s", num_cores=NC, num_subcores=NS)

_SC_PARAMS = pltpu.CompilerParams(
    use_tc_tiling_on_sc=False, needs_layout_passes=False)

_GATHER_DNUMS = lax.GatherDimensionNumbers(
    offset_dims=(), collapsed_slice_dims=(0,), start_index_map=(0,))


def _lane_perm(x, idx):
    return lax.gather(x, idx[:, None], _GATHER_DNUMS, (1,),
                      mode=lax.GatherScatterMode.PROMISE_IN_BOUNDS)


RPT = 624
TAIL = N - RPT * NS


def _zero_accum(zeros_hbm, accum, s):
    pltpu.sync_copy(zeros_hbm.at[pl.ds(s * RPT, RPT)],
                    accum.at[pl.ds(s * RPT, RPT)])

    @pl.when(s == 0)
    def _():
        pltpu.sync_copy(zeros_hbm.at[pl.ds(RPT * NS, TAIL)],
                        accum.at[pl.ds(RPT * NS, TAIL)])


def _drain_accum(accum, out_hbm, c, s):
    pltpu.sync_copy(accum.at[pl.ds(s * RPT, RPT)],
                    out_hbm.at[c, pl.ds(s * RPT, RPT)])

    @pl.when(s == 0)
    def _():
        pltpu.sync_copy(accum.at[pl.ds(RPT * NS, TAIL)],
                        out_hbm.at[c, pl.ds(RPT * NS, TAIL)])


def _copy_idx(src16, dst16):
    for g in range(CH // 16):
        dst16[pl.ds(g * 16, 16)] = src16[pl.ds(g * 16, 16)]


def _edge1_body(xl_hbm, xr_hbm, ei_hbm, attf_hbm, zeros_hbm, out_hbm, *scr):
    (src_i, dst_i, sidx, xl_r, xr_r, msg) = (scr[4*k:4*k+4] for k in range(6))
    attf_v = scr[24]
    accum = scr[25]
    sem_i = scr[26:30]
    sem_g = scr[30:34]
    sem_s = scr[34:38]
    c = lax.axis_index("c")
    s = lax.axis_index("s")
    wid = s * NC + c
    base = wid * EPW

    _zero_accum(zeros_hbm, accum, s)
    pltpu.sync_copy(attf_hbm, attf_v)
    plsc.subcore_barrier()

    attf = attf_v[...]
    lane = lax.iota(jnp.int32, 16)
    perm1 = lane ^ 1
    perm2 = lane ^ 2

    def issue_idx(ci, p):
        off = base + ci * CH
        pltpu.async_copy(ei_hbm.at[0, pl.ds(off, CH)], src_i[p], sem_i[p])
        pltpu.async_copy(ei_hbm.at[1, pl.ds(off, CH)], dst_i[p], sem_i[p])

    def wait_idx(p):
        pltpu.make_async_copy(ei_hbm.at[0, pl.ds(0, CH)], src_i[p],
                              sem_i[p]).wait()
        pltpu.make_async_copy(ei_hbm.at[1, pl.ds(0, CH)], dst_i[p],
                              sem_i[p]).wait()

    def issue_gather(p):
        pltpu.async_copy(xl_hbm.at[src_i[p]], xl_r[p], sem_g[p])
        pltpu.async_copy(xr_hbm.at[dst_i[p]], xr_r[p], sem_g[p])

    def wait_gather(p):
        pltpu.make_async_copy(xl_hbm.at[pl.ds(0, CH)], xl_r[p],
                              sem_g[p]).wait()
        pltpu.make_async_copy(xr_hbm.at[pl.ds(0, CH)], xr_r[p],
                              sem_g[p]).wait()

    def compute(p):
        _copy_idx(dst_i[p], sidx[p])

        @plsc.parallel_loop(0, CH, 1, unroll=16)
        def _(k):
            a = xl_r[p][k]
            b = xr_r[p][k]
            e = a + b
            e = jnp.maximum(e, 0.2 * e)
            w = e * attf
            w = w + _lane_perm(w, perm1)
            w = w + _lane_perm(w, perm2)
            pr = jnp.exp(w)
            msg[p][k, 0:16] = a * pr
            msg[p][k, 16:32] = pr

    def issue_scatter(p):
        pltpu.async_copy(msg[p], accum.at[sidx[p]], sem_s[p], add=True)

    def wait_scatter(p):
        pltpu.make_async_copy(msg[p], accum.at[pl.ds(0, CH)],
                              sem_s[p]).wait()

    for b in range(4):
        issue_idx(b, b)
    wait_idx(0)
    wait_idx(1)
    issue_gather(0)
    issue_gather(1)

    NQ = (NCHUNK - 1) // 4

    def quad(j, carry):
        for b in range(4):
            i = 4 * j + b
            p = b
            p2 = (b + 2) % 4
            wait_gather(p)

            @pl.when(j > 0)
            def _():
                wait_scatter(p)

            compute(p)
            issue_scatter(p)
            if b == 3:
                @pl.when(j < NQ - 1)
                def _():
                    wait_idx(p2)
                    issue_gather(p2)
                    issue_idx(i + 4, p)
            elif b in (1, 2):
                wait_idx(p2)
                issue_gather(p2)

                @pl.when(j < NQ - 1)
                def _():
                    issue_idx(i + 4, p)
            else:
                wait_idx(p2)
                issue_gather(p2)
                issue_idx(i + 4, p)
        return carry

    lax.fori_loop(0, NQ, quad, 0)

    wait_gather(0)
    wait_scatter(0)
    compute(0)
    issue_scatter(0)
    for b in (1, 2, 3, 0):
        wait_scatter(b)

    plsc.subcore_barrier()
    _drain_accum(accum, out_hbm, c, s)


_edge1 = functools.partial(
    pl.kernel,
    out_type=jax.ShapeDtypeStruct((NC, N, 2 * F1), jnp.float32),
    mesh=_mesh,
    compiler_params=_SC_PARAMS,
    scratch_types=(
        [pltpu.VMEM((CH,), jnp.int32)] * 12
        + [pltpu.VMEM((CH, F1), jnp.float32)] * 8
        + [pltpu.VMEM((CH, 2 * F1), jnp.float32)] * 4
        + [pltpu.VMEM((16,), jnp.float32),
           pltpu.VMEM_SHARED((N, 2 * F1), jnp.float32)]
        + [pltpu.SemaphoreType.DMA] * 12
    ),
)(_edge1_body)


def _edge2_body(tab_hbm, ei_hbm, att2_hbm, zeros_hbm, out_hbm, *scr):
    (src_i, dst_i, sidx, msg) = (scr[4*k:4*k+4] for k in range(4))
    tab = scr[16]
    att2_v = scr[17]
    accum = scr[18]
    sem_i = scr[19:23]
    sem_s = scr[23:27]
    c = lax.axis_index("c")
    s = lax.axis_index("s")
    wid = s * NC + c
    base = wid * EPW

    _zero_accum(zeros_hbm, accum, s)
    pltpu.sync_copy(tab_hbm, tab)
    pltpu.sync_copy(att2_hbm, att2_v)
    for b in range(4):
        pltpu.sync_copy(zeros_hbm.at[pl.ds(0, CH)], msg[b])
    plsc.subcore_barrier()

    att2 = att2_v[...]
    lane = lax.iota(jnp.int32, 16)
    zi = lane * 0
    oi = zi + 1

    def issue_idx(ci, p):
        off = base + ci * CH
        pltpu.async_copy(ei_hbm.at[0, pl.ds(off, CH)], src_i[p], sem_i[p])
        pltpu.async_copy(ei_hbm.at[1, pl.ds(off, CH)], dst_i[p], sem_i[p])

    def wait_idx(p):
        pltpu.make_async_copy(ei_hbm.at[0, pl.ds(0, CH)], src_i[p],
                              sem_i[p]).wait()
        pltpu.make_async_copy(ei_hbm.at[1, pl.ds(0, CH)], dst_i[p],
                              sem_i[p]).wait()

    def compute(p):
        _copy_idx(dst_i[p], sidx[p])

        @plsc.parallel_loop(0, CH // 16, 1, unroll=CH // 16)
        def _(g):
            g16 = g * 16
            sg = src_i[p][pl.ds(g16, 16)]
            dg = dst_i[p][pl.ds(g16, 16)]
            a = plsc.load_gather(tab, [sg, zi])
            b = plsc.load_gather(tab, [dg, oi])
            e = a + b
            e = jnp.maximum(e, 0.2 * e)
            pr = jnp.exp(e * att2)
            rows = g16 + lane
            plsc.store_scatter(msg[p], [rows, zi], pr * a)
            plsc.store_scatter(msg[p], [rows, oi], pr)

    def issue_scatter(p):
        pltpu.async_copy(msg[p], accum.at[sidx[p]], sem_s[p], add=True)

    def wait_scatter(p):
        pltpu.make_async_copy(msg[p], accum.at[pl.ds(0, CH)],
                              sem_s[p]).wait()

    for b in range(4):
        issue_idx(b, b)

    NQ = (NCHUNK - 1) // 4

    def quad(j, carry):
        for b in range(4):
            i = 4 * j + b
            p = b
            wait_idx(p)

            @pl.when(j > 0)
            def _():
                wait_scatter(p)

            compute(p)
            issue_scatter(p)
            if b == 0:
                issue_idx(i + 4, p)
            else:
                @pl.when(j < NQ - 1)
                def _():
                    issue_idx(i + 4, p)
        return carry

    lax.fori_loop(0, NQ, quad, 0)

    wait_idx(0)
    wait_scatter(0)
    compute(0)
    issue_scatter(0)
    for b in (1, 2, 3, 0):
        wait_scatter(b)

    plsc.subcore_barrier()
    _drain_accum(accum, out_hbm, c, s)


_edge2 = functools.partial(
    pl.kernel,
    out_type=jax.ShapeDtypeStruct((NC, N, 16), jnp.float32),
    mesh=_mesh,
    compiler_params=_SC_PARAMS,
    scratch_types=(
        [pltpu.VMEM((CH,), jnp.int32)] * 12
        + [pltpu.VMEM((CH, 16), jnp.float32)] * 4
        + [pltpu.VMEM((N, 2), jnp.float32),
           pltpu.VMEM((16,), jnp.float32),
           pltpu.VMEM_SHARED((N, 16), jnp.float32)]
        + [pltpu.SemaphoreType.DMA] * 8
    ),
)(_edge2_body)


def _mm1_body(x_ref, wl_ref, wr_ref, xl_ref, xr_ref):
    x = x_ref[...]
    xl_ref[...] = jnp.dot(x, wl_ref[...], preferred_element_type=jnp.float32)
    xr_ref[...] = jnp.dot(x, wr_ref[...], preferred_element_type=jnp.float32)


def _mid_body(p_ref, w2b_ref, b1t_ref, out_ref):
    acc = p_ref[0] + p_ref[1]
    den_sh = jnp.concatenate([acc[:, 16:], acc[:, :16]], axis=1)
    h = acc / (den_sh + 1e-16) + b1t_ref[...]
    h = jnp.where(h > 0, h, jnp.exp(h) - 1.0)
    out_ref[...] = jnp.dot(h, w2b_ref[...], preferred_element_type=jnp.float32)


def _fin_body(p2_ref, b2_ref, out_ref):
    acc = p2_ref[0] + p2_ref[1]
    den_sh = jnp.concatenate([acc[:, 1:], acc[:, :1]], axis=1)
    out_ref[...] = jax.nn.sigmoid(acc / (den_sh + 1e-16) + b2_ref[...])


def kernel(x, edge_index, W1l, W1r, att1, b1, W2l, W2r, att2, b2):
    xl1, xr1 = pl.pallas_call(
        _mm1_body,
        out_shape=[jax.ShapeDtypeStruct((N, F1), jnp.float32),
                   jax.ShapeDtypeStruct((N, F1), jnp.float32)],
    )(x, W1l, W1r)

    attf = att1.reshape(F1)
    zeros32 = jnp.zeros((N, 2 * F1), jnp.float32)
    part1 = _edge1(xl1, xr1, edge_index, attf, zeros32)

    w2cat = jnp.concatenate([W2l, W2r], axis=1)
    w2b = jnp.zeros((4, 32, 4, 2), jnp.float32)
    w2b = w2b.at[jnp.arange(4), :16, jnp.arange(4), :].set(w2cat)
    w2b = w2b.reshape(128, 8)
    b1t = jnp.tile(jnp.concatenate([b1, jnp.zeros(F1, jnp.float32)]), 4)

    xlr2p = pl.pallas_call(
        _mid_body,
        out_shape=jax.ShapeDtypeStruct((N // 4, 8), jnp.float32),
    )(part1.reshape(NC, N // 4, 128), w2b, b1t.reshape(1, 128))

    att2f = jnp.broadcast_to(att2.reshape(1, 1), (1, 16)).reshape(16)
    zeros16 = jnp.zeros((N, 16), jnp.float32)
    part2 = _edge2(xlr2p.reshape(N, 2), edge_index, att2f, zeros16)

    outp = pl.pallas_call(
        _fin_body,
        out_shape=jax.ShapeDtypeStruct((N // 8, 128), jnp.float32),
    )(part2.reshape(NC, N // 8, 128), b2.reshape(1, 1))
    return outp.reshape(N, 16)[:, 0:1]


# speedup vs baseline: 1.0004x; 1.0004x over previous
import functools

import jax
import jax.numpy as jnp
from jax import lax
from jax.experimental import pallas as pl
from jax.experimental.pallas import tpu as pltpu
from jax.experimental.pallas import tpu_sc as plsc

N = 10000
E = 320000
D = 128
F1 = 16
NC = 2
NS = 16
NW = NC * NS
EPW = E // NW
CH = 80
NCHUNK = EPW // CH
NPAIR = (NCHUNK - 1) // 2

@functools.lru_cache(maxsize=None)
def _get_mesh():
    return plsc.VectorSubcoreMesh(
        core_axis_name="c", subcore_axis_name="s",
        num_cores=NC, num_subcores=NS)

_SC_PARAMS = pltpu.CompilerParams(
    use_tc_tiling_on_sc=False, needs_layout_passes=False)

_GATHER_DNUMS = lax.GatherDimensionNumbers(
    offset_dims=(), collapsed_slice_dims=(0,), start_index_map=(0,))


def _lane_perm(x, idx):
    return lax.gather(x, idx[:, None], _GATHER_DNUMS, (1,),
                      mode=lax.GatherScatterMode.PROMISE_IN_BOUNDS)


RPT = 624
TAIL = N - RPT * NS


def _zero_accum(zeros_hbm, accum, s):
    pltpu.sync_copy(zeros_hbm.at[pl.ds(s * RPT, RPT)],
                    accum.at[pl.ds(s * RPT, RPT)])

    @pl.when(s == 0)
    def _():
        pltpu.sync_copy(zeros_hbm.at[pl.ds(RPT * NS, TAIL)],
                        accum.at[pl.ds(RPT * NS, TAIL)])


def _drain_accum(accum, out_hbm, c, s):
    pltpu.sync_copy(accum.at[pl.ds(s * RPT, RPT)],
                    out_hbm.at[c, pl.ds(s * RPT, RPT)])

    @pl.when(s == 0)
    def _():
        pltpu.sync_copy(accum.at[pl.ds(RPT * NS, TAIL)],
                        out_hbm.at[c, pl.ds(RPT * NS, TAIL)])


def _copy_idx(src16, dst16):
    for g in range(CH // 16):
        dst16[pl.ds(g * 16, 16)] = src16[pl.ds(g * 16, 16)]


def _edge1_body(xl_hbm, xr_hbm, ei_hbm, attf_hbm, zeros_hbm, out_hbm, *scr):
    (src_i, dst_i, sidx, xl_r, xr_r, msg) = (scr[4*k:4*k+4] for k in range(6))
    attf_v = scr[24]
    accum = scr[25]
    sem_i = scr[26:30]
    sem_g = scr[30:34]
    sem_s = scr[34:38]
    c = lax.axis_index("c")
    s = lax.axis_index("s")
    wid = s * NC + c
    base = wid * EPW

    _zero_accum(zeros_hbm, accum, s)
    pltpu.sync_copy(attf_hbm, attf_v)
    plsc.subcore_barrier()

    attf = attf_v[...]
    lane = lax.iota(jnp.int32, 16)
    perm1 = lane ^ 1
    perm2 = lane ^ 2

    def issue_idx(ci, p):
        off = base + ci * CH
        pltpu.async_copy(ei_hbm.at[0, pl.ds(off, CH)], src_i[p], sem_i[p])
        pltpu.async_copy(ei_hbm.at[1, pl.ds(off, CH)], dst_i[p], sem_i[p])

    def wait_idx(p):
        pltpu.make_async_copy(ei_hbm.at[0, pl.ds(0, CH)], src_i[p],
                              sem_i[p]).wait()
        pltpu.make_async_copy(ei_hbm.at[1, pl.ds(0, CH)], dst_i[p],
                              sem_i[p]).wait()

    def issue_gather(p):
        pltpu.async_copy(xl_hbm.at[src_i[p]], xl_r[p], sem_g[p])
        pltpu.async_copy(xr_hbm.at[dst_i[p]], xr_r[p], sem_g[p])

    def wait_gather(p):
        pltpu.make_async_copy(xl_hbm.at[pl.ds(0, CH)], xl_r[p],
                              sem_g[p]).wait()
        pltpu.make_async_copy(xr_hbm.at[pl.ds(0, CH)], xr_r[p],
                              sem_g[p]).wait()

    def compute(p):
        _copy_idx(dst_i[p], sidx[p])

        @plsc.parallel_loop(0, CH, 1, unroll=16)
        def _(k):
            a = xl_r[p][k]
            b = xr_r[p][k]
            e = a + b
            e = jnp.maximum(e, 0.2 * e)
            w = e * attf
            w = w + _lane_perm(w, perm1)
            w = w + _lane_perm(w, perm2)
            pr = jnp.exp(w)
            msg[p][k, 0:16] = a * pr
            msg[p][k, 16:32] = pr

    def issue_scatter(p):
        pltpu.async_copy(msg[p], accum.at[sidx[p]], sem_s[p], add=True)

    def wait_scatter(p):
        pltpu.make_async_copy(msg[p], accum.at[pl.ds(0, CH)],
                              sem_s[p]).wait()

    for b in range(4):
        issue_idx(b, b)
    wait_idx(0)
    wait_idx(1)
    issue_gather(0)
    issue_gather(1)

    NQ = (NCHUNK - 1) // 4

    def quad(j, carry):
        for b in range(4):
            i = 4 * j + b
            p = b
            p2 = (b + 2) % 4
            wait_gather(p)

            @pl.when(j > 0)
            def _():
                wait_scatter(p)

            compute(p)
            issue_scatter(p)
            if b == 3:
                @pl.when(j < NQ - 1)
                def _():
                    wait_idx(p2)
                    issue_gather(p2)
                    issue_idx(i + 4, p)
            elif b in (1, 2):
                wait_idx(p2)
                issue_gather(p2)

                @pl.when(j < NQ - 1)
                def _():
                    issue_idx(i + 4, p)
            else:
                wait_idx(p2)
                issue_gather(p2)
                issue_idx(i + 4, p)
        return carry

    lax.fori_loop(0, NQ, quad, 0)

    wait_gather(0)
    wait_scatter(0)
    compute(0)
    issue_scatter(0)
    for b in (1, 2, 3, 0):
        wait_scatter(b)

    plsc.subcore_barrier()
    _drain_accum(accum, out_hbm, c, s)


@functools.lru_cache(maxsize=None)
def _edge1():
    return functools.partial(
        pl.kernel,
        out_type=jax.ShapeDtypeStruct((NC, N, 2 * F1), jnp.float32),
        mesh=_get_mesh(),
    compiler_params=_SC_PARAMS,
    scratch_types=(
        [pltpu.VMEM((CH,), jnp.int32)] * 12
        + [pltpu.VMEM((CH, F1), jnp.float32)] * 8
        + [pltpu.VMEM((CH, 2 * F1), jnp.float32)] * 4
        + [pltpu.VMEM((16,), jnp.float32),
           pltpu.VMEM_SHARED((N, 2 * F1), jnp.float32)]
            + [pltpu.SemaphoreType.DMA] * 12
        ),
    )(_edge1_body)


def _edge2_body(tab_hbm, ei_hbm, att2_hbm, zeros_hbm, out_hbm, *scr):
    (src_i, dst_i, sidx, msg) = (scr[4*k:4*k+4] for k in range(4))
    tab = scr[16]
    att2_v = scr[17]
    accum = scr[18]
    sem_i = scr[19:23]
    sem_s = scr[23:27]
    c = lax.axis_index("c")
    s = lax.axis_index("s")
    wid = s * NC + c
    base = wid * EPW

    _zero_accum(zeros_hbm, accum, s)
    pltpu.sync_copy(tab_hbm, tab)
    pltpu.sync_copy(att2_hbm, att2_v)
    for b in range(4):
        pltpu.sync_copy(zeros_hbm.at[pl.ds(0, CH)], msg[b])
    plsc.subcore_barrier()

    att2 = att2_v[...]
    lane = lax.iota(jnp.int32, 16)
    zi = lane * 0
    oi = zi + 1

    def issue_idx(ci, p):
        off = base + ci * CH
        pltpu.async_copy(ei_hbm.at[0, pl.ds(off, CH)], src_i[p], sem_i[p])
        pltpu.async_copy(ei_hbm.at[1, pl.ds(off, CH)], dst_i[p], sem_i[p])

    def wait_idx(p):
        pltpu.make_async_copy(ei_hbm.at[0, pl.ds(0, CH)], src_i[p],
                              sem_i[p]).wait()
        pltpu.make_async_copy(ei_hbm.at[1, pl.ds(0, CH)], dst_i[p],
                              sem_i[p]).wait()

    def compute(p):
        _copy_idx(dst_i[p], sidx[p])

        @plsc.parallel_loop(0, CH // 16, 1, unroll=CH // 16)
        def _(g):
            g16 = g * 16
            sg = src_i[p][pl.ds(g16, 16)]
            dg = dst_i[p][pl.ds(g16, 16)]
            a = plsc.load_gather(tab, [sg, zi])
            b = plsc.load_gather(tab, [dg, oi])
            e = a + b
            e = jnp.maximum(e, 0.2 * e)
            pr = jnp.exp(e * att2)
            rows = g16 + lane
            plsc.store_scatter(msg[p], [rows, zi], pr * a)
            plsc.store_scatter(msg[p], [rows, oi], pr)

    def issue_scatter(p):
        pltpu.async_copy(msg[p], accum.at[sidx[p]], sem_s[p], add=True)

    def wait_scatter(p):
        pltpu.make_async_copy(msg[p], accum.at[pl.ds(0, CH)],
                              sem_s[p]).wait()

    for b in range(4):
        issue_idx(b, b)

    NQ = (NCHUNK - 1) // 4

    def quad(j, carry):
        for b in range(4):
            i = 4 * j + b
            p = b
            wait_idx(p)

            @pl.when(j > 0)
            def _():
                wait_scatter(p)

            compute(p)
            issue_scatter(p)
            if b == 0:
                issue_idx(i + 4, p)
            else:
                @pl.when(j < NQ - 1)
                def _():
                    issue_idx(i + 4, p)
        return carry

    lax.fori_loop(0, NQ, quad, 0)

    wait_idx(0)
    wait_scatter(0)
    compute(0)
    issue_scatter(0)
    for b in (1, 2, 3, 0):
        wait_scatter(b)

    plsc.subcore_barrier()
    _drain_accum(accum, out_hbm, c, s)


@functools.lru_cache(maxsize=None)
def _edge2():
    return functools.partial(
        pl.kernel,
        out_type=jax.ShapeDtypeStruct((NC, N, 16), jnp.float32),
        mesh=_get_mesh(),
    compiler_params=_SC_PARAMS,
    scratch_types=(
        [pltpu.VMEM((CH,), jnp.int32)] * 12
        + [pltpu.VMEM((CH, 16), jnp.float32)] * 4
        + [pltpu.VMEM((N, 2), jnp.float32),
           pltpu.VMEM((16,), jnp.float32),
           pltpu.VMEM_SHARED((N, 16), jnp.float32)]
            + [pltpu.SemaphoreType.DMA] * 8
        ),
    )(_edge2_body)


def _mm1_body(x_ref, wl_ref, wr_ref, xl_ref, xr_ref):
    x = x_ref[...]
    xl_ref[...] = jnp.dot(x, wl_ref[...], preferred_element_type=jnp.float32)
    xr_ref[...] = jnp.dot(x, wr_ref[...], preferred_element_type=jnp.float32)


def _mid_body(p_ref, w2b_ref, b1t_ref, out_ref):
    acc = p_ref[0] + p_ref[1]
    den_sh = jnp.concatenate([acc[:, 16:], acc[:, :16]], axis=1)
    h = acc / (den_sh + 1e-16) + b1t_ref[...]
    h = jnp.where(h > 0, h, jnp.exp(h) - 1.0)
    out_ref[...] = jnp.dot(h, w2b_ref[...], preferred_element_type=jnp.float32)


def _fin_body(p2_ref, b2_ref, out_ref):
    acc = p2_ref[0] + p2_ref[1]
    den_sh = jnp.concatenate([acc[:, 1:], acc[:, :1]], axis=1)
    out_ref[...] = jax.nn.sigmoid(acc / (den_sh + 1e-16) + b2_ref[...])


def kernel(x, edge_index, W1l, W1r, att1, b1, W2l, W2r, att2, b2):
    xl1, xr1 = pl.pallas_call(
        _mm1_body,
        out_shape=[jax.ShapeDtypeStruct((N, F1), jnp.float32),
                   jax.ShapeDtypeStruct((N, F1), jnp.float32)],
    )(x, W1l, W1r)

    attf = att1.reshape(F1)
    zeros32 = jnp.zeros((N, 2 * F1), jnp.float32)
    part1 = _edge1()(xl1, xr1, edge_index, attf, zeros32)

    w2cat = jnp.concatenate([W2l, W2r], axis=1)
    w2b = jnp.zeros((4, 32, 4, 2), jnp.float32)
    w2b = w2b.at[jnp.arange(4), :16, jnp.arange(4), :].set(w2cat)
    w2b = w2b.reshape(128, 8)
    b1t = jnp.tile(jnp.concatenate([b1, jnp.zeros(F1, jnp.float32)]), 4)

    xlr2p = pl.pallas_call(
        _mid_body,
        out_shape=jax.ShapeDtypeStruct((N // 4, 8), jnp.float32),
    )(part1.reshape(NC, N // 4, 128), w2b, b1t.reshape(1, 128))

    att2f = jnp.broadcast_to(att2.reshape(1, 1), (1, 16)).reshape(16)
    zeros16 = jnp.zeros((N, 16), jnp.float32)
    part2 = _edge2()(xlr2p.reshape(N, 2), edge_index, att2f, zeros16)

    outp = pl.pallas_call(
        _fin_body,
        out_shape=jax.ShapeDtypeStruct((N // 8, 128), jnp.float32),
    )(part2.reshape(NC, N // 8, 128), b2.reshape(1, 1))
    return outp.reshape(N, 16)[:, 0:1]
